# baseline (speedup 1.0000x reference)
import functools
import math

import jax
import jax.numpy as jnp
from jax import lax
from jax.experimental import pallas as pl
from jax.experimental.pallas import tpu as pltpu
from jax.experimental.pallas import tpu_sc as plsc

D_MODEL = 64
SCALE = math.sqrt(D_MODEL)
LANES = 16


@functools.cache
def _build(B):
    info = plsc.get_sparse_core_info()
    NC, NS = info.num_cores, info.num_subcores
    NW = NC * NS
    assert B % NW == 0
    b_per_w = B // NW
    C = 128
    assert b_per_w % C == 0
    n_chunks = b_per_w // C

    mesh = plsc.VectorSubcoreMesh(core_axis_name="c", subcore_axis_name="s")

    @functools.partial(
        pl.kernel,
        mesh=mesh,
        out_type=jax.ShapeDtypeStruct((B, D_MODEL), jnp.float32),
        scratch_types=[
            pltpu.VMEM((C,), jnp.int32),
            pltpu.VMEM((C, D_MODEL), jnp.float32),
            pltpu.SemaphoreType.DMA,
        ],
        compiler_params=pltpu.CompilerParams(use_tc_tiling_on_sc=False),
    )
    def emb_kernel(x_hbm, lut_hbm, out_hbm, idx_v, rows_v, sem):
        wid = lax.axis_index("s") * NC + lax.axis_index("c")
        base = wid * b_per_w

        def chunk(g, carry):
            off = base + g * C
            pltpu.sync_copy(x_hbm.at[pl.ds(off, C)], idx_v)
            pltpu.async_copy(lut_hbm.at[idx_v], rows_v, sem).wait()

            def row(i, c):
                for j in range(D_MODEL // LANES):
                    s = pl.ds(j * LANES, LANES)
                    rows_v[i, s] = rows_v[i, s] * SCALE
                return c

            lax.fori_loop(0, C, row, 0)
            pltpu.sync_copy(rows_v, out_hbm.at[pl.ds(off, C)])
            return carry

        lax.fori_loop(0, n_chunks, chunk, 0)

    return emb_kernel


def kernel(x, lut):
    orig_shape = x.shape
    xf = jnp.reshape(x, (-1,)).astype(jnp.int32)
    out = _build(xf.shape[0])(xf, lut)
    return jnp.reshape(out, orig_shape + (D_MODEL,))


# speedup vs baseline: 1.2813x; 1.2813x over previous
import functools
import math

import jax
import jax.numpy as jnp
from jax import lax
from jax.experimental import pallas as pl
from jax.experimental.pallas import tpu as pltpu
from jax.experimental.pallas import tpu_sc as plsc

D_MODEL = 64
SCALE = math.sqrt(D_MODEL)
LANES = 16
C = 128
NBUF = 4


@functools.cache
def _build(B):
    info = plsc.get_sparse_core_info()
    NC, NS = info.num_cores, info.num_subcores
    NW = NC * NS
    assert B % (NW * C) == 0
    b_per_w = B // NW
    n_chunks = b_per_w // C
    assert n_chunks % NBUF == 0

    mesh = plsc.VectorSubcoreMesh(core_axis_name="c", subcore_axis_name="s")

    @functools.partial(
        pl.kernel,
        mesh=mesh,
        out_type=jax.ShapeDtypeStruct((B, D_MODEL), jnp.float32),
        scratch_types=[
            pltpu.VMEM((n_chunks, C), jnp.int32),
        ]
        + [pltpu.VMEM((C, D_MODEL), jnp.float32) for _ in range(2 * NBUF)]
        + [pltpu.SemaphoreType.DMA for _ in range(2 * NBUF)],
        compiler_params=pltpu.CompilerParams(use_tc_tiling_on_sc=False),
    )
    def emb_kernel(x2_hbm, lut_hbm, out_hbm, idx_v, *bufs_and_sems):
        in_bufs = bufs_and_sems[0:NBUF]
        out_bufs = bufs_and_sems[NBUF : 2 * NBUF]
        gsems = bufs_and_sems[2 * NBUF : 3 * NBUF]
        ssems = bufs_and_sems[3 * NBUF : 4 * NBUF]

        wid = lax.axis_index("s") * NC + lax.axis_index("c")
        cbase = wid * n_chunks

        pltpu.sync_copy(x2_hbm.at[pl.ds(cbase, n_chunks)], idx_v)

        def gather(b, g):
            return pltpu.make_async_copy(
                lut_hbm.at[idx_v.at[g]], in_bufs[b], gsems[b]
            )

        def scatter(b, g):
            return pltpu.make_async_copy(
                out_bufs[b], out_hbm.at[pl.ds((cbase + g) * C, C)], ssems[b]
            )

        for b in range(NBUF):
            gather(b, b).start()

        @pl.loop(0, n_chunks, step=NBUF)
        def _(g0):
            for b in range(NBUF):
                g = g0 + b
                gather(b, g).wait()

                @pl.when(g >= NBUF)
                def _():
                    scatter(b, g - NBUF).wait()

                src = in_bufs[b]
                dst = out_bufs[b]

                @plsc.parallel_loop(0, C, unroll=8)
                def _(i):
                    for j in range(D_MODEL // LANES):
                        s = pl.ds(j * LANES, LANES)
                        dst[i, s] = src[i, s] * SCALE

                scatter(b, g).start()

                @pl.when(g + NBUF < n_chunks)
                def _():
                    gather(b, g + NBUF).start()

        for b in range(NBUF):
            scatter(b, n_chunks - NBUF + b).wait()

    return emb_kernel


def kernel(x, lut):
    orig_shape = x.shape
    xf = jnp.reshape(x, (-1, C)).astype(jnp.int32)
    out = _build(xf.size)(xf, lut)
    return jnp.reshape(out, orig_shape + (D_MODEL,))
